# Initial kernel scaffold; baseline (speedup 1.0000x reference)
import jax
import jax.numpy as jnp
from jax.experimental import pallas as pl


def kernel(x, edge_index, W_mu, b_mu, W_lv, b_lv):
    raise NotImplementedError("write your pallas kernel here")


# speedup vs baseline: 3.2454x; 3.2454x over previous
import functools

import jax
import jax.numpy as jnp
from jax import lax
from jax.experimental import pallas as pl
from jax.experimental.pallas import tpu as pltpu
from jax.experimental.pallas import tpu_sc as plsc

N_NODES = 10000
N_EDGES = 320000
D_FEAT = 128
HIDDEN = 64

NC = 2
NS = 16
NW = NC * NS
HALF = D_FEAT // NC
EPW = N_EDGES // NW
EPT = N_EDGES // NS
K = 80
NCHUNK = EPW // K
NCHUNK_E = EPT // K
STRIPE = 640
TAIL_STRIPE = N_NODES - (NS - 1) * STRIPE
DEGW = 16

_mesh = plsc.VectorSubcoreMesh(core_axis_name="c", subcore_axis_name="s")


@functools.partial(
    pl.kernel,
    out_type=(
        jax.ShapeDtypeStruct((NC, N_NODES, HALF), jnp.float32),
        jax.ShapeDtypeStruct((N_NODES, DEGW), jnp.float32),
    ),
    mesh=_mesh,
    scratch_types=(
        pltpu.VMEM((NCHUNK_E, K), jnp.int32),
        pltpu.VMEM((NCHUNK_E, K), jnp.int32),
        pltpu.VMEM((K, HALF), jnp.float32),
        pltpu.VMEM((K, DEGW), jnp.float32),
        pltpu.VMEM_SHARED((N_NODES, HALF), jnp.float32),
        pltpu.VMEM_SHARED((N_NODES, DEGW), jnp.float32),
        pltpu.SemaphoreType.DMA,
    ),
    compiler_params=pltpu.CompilerParams(use_tc_tiling_on_sc=False, needs_layout_passes=False),
)
def _encoder(x2_hbm, src_hbm, dst_hbm, zfeat_hbm, zdeg_hbm, ones_hbm,
             agg_out, deg_out,
             srcv, dstv, rows, ones, agg_sh, deg_sh, sem):
    c = lax.axis_index("c")
    s = lax.axis_index("s")
    r0 = pl.multiple_of(s * STRIPE, STRIPE)

    @pl.when(s < NS - 1)
    def _zero_main():
        pltpu.sync_copy(zfeat_hbm.at[pl.ds(r0, STRIPE)],
                        agg_sh.at[pl.ds(r0, STRIPE)])
        pltpu.sync_copy(zdeg_hbm.at[pl.ds(r0, STRIPE)],
                        deg_sh.at[pl.ds(r0, STRIPE)])

    @pl.when(s == NS - 1)
    def _zero_tail():
        pltpu.sync_copy(zfeat_hbm.at[pl.ds((NS - 1) * STRIPE, TAIL_STRIPE)],
                        agg_sh.at[pl.ds((NS - 1) * STRIPE, TAIL_STRIPE)])
        pltpu.sync_copy(zdeg_hbm.at[pl.ds((NS - 1) * STRIPE, TAIL_STRIPE)],
                        deg_sh.at[pl.ds((NS - 1) * STRIPE, TAIL_STRIPE)])

    pltpu.sync_copy(ones_hbm, ones)
    pltpu.sync_copy(src_hbm.at[s], srcv)
    pltpu.sync_copy(dst_hbm.at[s], dstv)
    plsc.subcore_barrier()

    @pl.loop(0, NCHUNK_E)
    def _chunk(ci):
        pltpu.async_copy(x2_hbm.at[c].at[srcv.at[ci]], rows, sem).wait()
        pltpu.sync_copy(rows, agg_sh.at[dstv.at[ci]], add=True)

        @pl.when(c == 0)
        def _deg():
            pltpu.sync_copy(ones, deg_sh.at[dstv.at[ci]], add=True)

    plsc.subcore_barrier()

    @pl.when(s < NS - 1)
    def _out_main():
        pltpu.sync_copy(agg_sh.at[pl.ds(r0, STRIPE)],
                        agg_out.at[c, pl.ds(r0, STRIPE)])

        @pl.when(c == 0)
        def _dmain():
            pltpu.sync_copy(deg_sh.at[pl.ds(r0, STRIPE)],
                            deg_out.at[pl.ds(r0, STRIPE)])

    @pl.when(s == NS - 1)
    def _out_tail():
        pltpu.sync_copy(agg_sh.at[pl.ds((NS - 1) * STRIPE, TAIL_STRIPE)],
                        agg_out.at[c, pl.ds((NS - 1) * STRIPE, TAIL_STRIPE)])

        @pl.when(c == 0)
        def _dtail():
            pltpu.sync_copy(deg_sh.at[pl.ds((NS - 1) * STRIPE, TAIL_STRIPE)],
                            deg_out.at[pl.ds((NS - 1) * STRIPE, TAIL_STRIPE)])


def _dense_body(pa_ref, dg_ref, wmu_ref, bmu_ref, wlv_ref, blv_ref, eps_ref,
                mu_ref, lv_ref, z_ref):
    p = jnp.concatenate([pa_ref[0], pa_ref[1]], axis=-1)
    deg = dg_ref[...][:, 0:1]
    agg = p / jnp.maximum(deg, 1.0)
    mu = jnp.dot(agg, wmu_ref[...], preferred_element_type=jnp.float32) + bmu_ref[...]
    lv = jnp.dot(agg, wlv_ref[...], preferred_element_type=jnp.float32) + blv_ref[...]
    z = mu + eps_ref[...] * jnp.exp(0.5 * lv)
    mu_ref[...] = mu
    lv_ref[...] = lv
    z_ref[...] = z


_DENSE_BLK = 1000


def _dense(pa, dg, W_mu, b_mu, W_lv, b_lv, eps):
    n_blocks = N_NODES // _DENSE_BLK
    return pl.pallas_call(
        _dense_body,
        grid=(n_blocks,),
        in_specs=[
            pl.BlockSpec((NC, _DENSE_BLK, HALF), lambda i: (0, i, 0)),
            pl.BlockSpec((_DENSE_BLK, DEGW), lambda i: (i, 0)),
            pl.BlockSpec((D_FEAT, HIDDEN), lambda i: (0, 0)),
            pl.BlockSpec((1, HIDDEN), lambda i: (0, 0)),
            pl.BlockSpec((D_FEAT, HIDDEN), lambda i: (0, 0)),
            pl.BlockSpec((1, HIDDEN), lambda i: (0, 0)),
            pl.BlockSpec((_DENSE_BLK, HIDDEN), lambda i: (i, 0)),
        ],
        out_specs=[
            pl.BlockSpec((_DENSE_BLK, HIDDEN), lambda i: (i, 0)),
            pl.BlockSpec((_DENSE_BLK, HIDDEN), lambda i: (i, 0)),
            pl.BlockSpec((_DENSE_BLK, HIDDEN), lambda i: (i, 0)),
        ],
        out_shape=[
            jax.ShapeDtypeStruct((N_NODES, HIDDEN), jnp.float32),
            jax.ShapeDtypeStruct((N_NODES, HIDDEN), jnp.float32),
            jax.ShapeDtypeStruct((N_NODES, HIDDEN), jnp.float32),
        ],
    )(pa, dg, W_mu, b_mu, W_lv, b_lv, eps)


@functools.partial(
    pl.kernel,
    out_type=jax.ShapeDtypeStruct((N_EDGES,), jnp.float32),
    mesh=_mesh,
    scratch_types=(
        pltpu.VMEM((NCHUNK, K), jnp.int32),
        pltpu.VMEM((NCHUNK, K), jnp.int32),
        pltpu.VMEM((K, HIDDEN), jnp.float32),
        pltpu.VMEM((K, HIDDEN), jnp.float32),
        pltpu.VMEM((K,), jnp.float32),
        pltpu.SemaphoreType.DMA,
    ),
    compiler_params=pltpu.CompilerParams(use_tc_tiling_on_sc=False, needs_layout_passes=False),
)
def _decoder(z_hbm, src_hbm, dst_hbm, recon_out,
             srcv, dstv, zs, zd, outv, sem):
    c = lax.axis_index("c")
    s = lax.axis_index("s")
    w = s * NC + c
    base = w * EPW
    pltpu.sync_copy(src_hbm.at[w], srcv)
    pltpu.sync_copy(dst_hbm.at[w], dstv)

    @pl.loop(0, NCHUNK)
    def _chunk(ci):
        cs = pltpu.async_copy(z_hbm.at[srcv.at[ci]], zs, sem)
        cs.wait()
        cd = pltpu.async_copy(z_hbm.at[dstv.at[ci]], zd, sem)
        cd.wait()
        for g in range(K // 16):
            row = lax.iota(jnp.int32, 16) + (g * 16)
            acc = jnp.zeros((16,), jnp.float32)
            for dd in range(HIDDEN):
                col = jnp.full((16,), dd, jnp.int32)
                a = plsc.load_gather(zs, [row, col])
                b = plsc.load_gather(zd, [row, col])
                acc = acc + a * b
            outv[pl.ds(g * 16, 16)] = 1.0 / (1.0 + jnp.exp(-acc))
        off = pl.multiple_of(base + ci * K, 8)
        pltpu.sync_copy(outv, recon_out.at[pl.ds(off, K)])


def kernel(x, edge_index, W_mu, b_mu, W_lv, b_lv):
    ei = edge_index.astype(jnp.int32)
    src_e = ei[0].reshape(NS, NCHUNK_E, K)
    dst_e = ei[1].reshape(NS, NCHUNK_E, K)
    src_d = ei[0].reshape(NW, NCHUNK, K)
    dst_d = ei[1].reshape(NW, NCHUNK, K)
    x2 = x.reshape(N_NODES, NC, HALF).transpose(1, 0, 2)
    zfeat = jnp.zeros((N_NODES, HALF), jnp.float32)
    zdeg = jnp.zeros((N_NODES, DEGW), jnp.float32)
    ones = jnp.ones((K, DEGW), jnp.float32)
    pa, dg = _encoder(x2, src_e, dst_e, zfeat, zdeg, ones)
    eps = jax.random.normal(jax.random.key(42), (N_NODES, HIDDEN), jnp.float32)
    mu, lv, z = _dense(pa, dg, W_mu, b_mu.reshape(1, HIDDEN),
                       W_lv, b_lv.reshape(1, HIDDEN), eps)
    recon = _decoder(z, src_d, dst_d)
    return (recon, mu, lv, z)


# speedup vs baseline: 4.3007x; 1.3252x over previous
import functools

import jax
import jax.numpy as jnp
from jax import lax
from jax.experimental import pallas as pl
from jax.experimental.pallas import tpu as pltpu
from jax.experimental.pallas import tpu_sc as plsc

N_NODES = 10000
N_EDGES = 320000
D_FEAT = 128
HIDDEN = 64

NC = 2
NS = 16
NW = NC * NS
HALF = D_FEAT // NC
EPW = N_EDGES // NW
EPT = N_EDGES // NS
K = 80
NCHUNK = EPW // K
NCHUNK_E = EPT // K
STRIPE = 640
TAIL_STRIPE = N_NODES - (NS - 1) * STRIPE
DEGW = 16

_mesh = plsc.VectorSubcoreMesh(core_axis_name="c", subcore_axis_name="s")


@functools.partial(
    pl.kernel,
    out_type=(
        jax.ShapeDtypeStruct((NC, N_NODES, HALF), jnp.float32),
        jax.ShapeDtypeStruct((NC, N_NODES, DEGW), jnp.float32),
    ),
    mesh=_mesh,
    scratch_types=(
        pltpu.VMEM((NCHUNK_E, K), jnp.int32),
        pltpu.VMEM((NCHUNK_E, K), jnp.int32),
        pltpu.VMEM((K, HALF), jnp.float32),
        pltpu.VMEM((K, HALF), jnp.float32),
        pltpu.VMEM((K, DEGW), jnp.float32),
        pltpu.VMEM_SHARED((N_NODES, HALF), jnp.float32),
        pltpu.VMEM_SHARED((N_NODES, DEGW), jnp.float32),
        pltpu.SemaphoreType.DMA,
        pltpu.SemaphoreType.DMA,
        pltpu.SemaphoreType.DMA,
        pltpu.SemaphoreType.DMA,
    ),
    compiler_params=pltpu.CompilerParams(use_tc_tiling_on_sc=False, needs_layout_passes=False),
)
def _encoder(x2_hbm, src_hbm, dst_hbm, zfeat_hbm, zdeg_hbm, ones_hbm,
             agg_out, deg_out,
             srcv, dstv, rows0, rows1, ones, agg_sh, deg_sh,
             semg0, semg1, sems, semd):
    c = lax.axis_index("c")
    s = lax.axis_index("s")
    r0 = pl.multiple_of(s * STRIPE, STRIPE)

    @pl.when(s < NS - 1)
    def _zero_main():
        pltpu.sync_copy(zfeat_hbm.at[pl.ds(r0, STRIPE)],
                        agg_sh.at[pl.ds(r0, STRIPE)])
        pltpu.sync_copy(zdeg_hbm.at[pl.ds(r0, STRIPE)],
                        deg_sh.at[pl.ds(r0, STRIPE)])

    @pl.when(s == NS - 1)
    def _zero_tail():
        pltpu.sync_copy(zfeat_hbm.at[pl.ds((NS - 1) * STRIPE, TAIL_STRIPE)],
                        agg_sh.at[pl.ds((NS - 1) * STRIPE, TAIL_STRIPE)])
        pltpu.sync_copy(zdeg_hbm.at[pl.ds((NS - 1) * STRIPE, TAIL_STRIPE)],
                        deg_sh.at[pl.ds((NS - 1) * STRIPE, TAIL_STRIPE)])

    pltpu.sync_copy(ones_hbm, ones)
    pltpu.sync_copy(src_hbm.at[s], srcv)
    pltpu.sync_copy(dst_hbm.at[s], dstv)
    plsc.subcore_barrier()

    def _gather(ci, buf, sem):
        pltpu.async_copy(x2_hbm.at[c].at[srcv.at[ci]], buf, sem)

    def _gather_wait(ci, buf, sem):
        pltpu.make_async_copy(x2_hbm.at[c].at[srcv.at[ci]], buf, sem).wait()

    def _consume(ci, buf):
        a = pltpu.async_copy(buf, agg_sh.at[dstv.at[ci]], sems, add=True)

        @pl.when(lax.rem(ci, NC) == c)
        def _deg():
            pltpu.async_copy(ones, deg_sh.at[dstv.at[ci]], semd, add=True).wait()

        a.wait()

    _gather(0, rows0, semg0)

    @pl.loop(0, NCHUNK_E // 2)
    def _chunk(i):
        ci0 = i * 2
        ci1 = ci0 + 1
        _gather(ci1, rows1, semg1)
        _gather_wait(ci0, rows0, semg0)
        _consume(ci0, rows0)

        @pl.when(i < NCHUNK_E // 2 - 1)
        def _pref():
            _gather(ci0 + 2, rows0, semg0)

        _gather_wait(ci1, rows1, semg1)
        _consume(ci1, rows1)

    plsc.subcore_barrier()

    @pl.when(s < NS - 1)
    def _out_main():
        pltpu.sync_copy(agg_sh.at[pl.ds(r0, STRIPE)],
                        agg_out.at[c, pl.ds(r0, STRIPE)])
        pltpu.sync_copy(deg_sh.at[pl.ds(r0, STRIPE)],
                        deg_out.at[c, pl.ds(r0, STRIPE)])

    @pl.when(s == NS - 1)
    def _out_tail():
        pltpu.sync_copy(agg_sh.at[pl.ds((NS - 1) * STRIPE, TAIL_STRIPE)],
                        agg_out.at[c, pl.ds((NS - 1) * STRIPE, TAIL_STRIPE)])
        pltpu.sync_copy(deg_sh.at[pl.ds((NS - 1) * STRIPE, TAIL_STRIPE)],
                        deg_out.at[c, pl.ds((NS - 1) * STRIPE, TAIL_STRIPE)])


def _dense_body(pa_ref, dg_ref, wmu_ref, bmu_ref, wlv_ref, blv_ref, eps_ref,
                mu_ref, lv_ref, z_ref):
    p = jnp.concatenate([pa_ref[0], pa_ref[1]], axis=-1)
    deg = (dg_ref[0] + dg_ref[1])[:, 0:1]
    agg = p / jnp.maximum(deg, 1.0)
    mu = jnp.dot(agg, wmu_ref[...], preferred_element_type=jnp.float32) + bmu_ref[...]
    lv = jnp.dot(agg, wlv_ref[...], preferred_element_type=jnp.float32) + blv_ref[...]
    z = mu + eps_ref[...] * jnp.exp(0.5 * lv)
    mu_ref[...] = mu
    lv_ref[...] = lv
    z_ref[...] = z


_DENSE_BLK = 1000


def _dense(pa, dg, W_mu, b_mu, W_lv, b_lv, eps):
    n_blocks = N_NODES // _DENSE_BLK
    return pl.pallas_call(
        _dense_body,
        grid=(n_blocks,),
        in_specs=[
            pl.BlockSpec((NC, _DENSE_BLK, HALF), lambda i: (0, i, 0)),
            pl.BlockSpec((NC, _DENSE_BLK, DEGW), lambda i: (0, i, 0)),
            pl.BlockSpec((D_FEAT, HIDDEN), lambda i: (0, 0)),
            pl.BlockSpec((1, HIDDEN), lambda i: (0, 0)),
            pl.BlockSpec((D_FEAT, HIDDEN), lambda i: (0, 0)),
            pl.BlockSpec((1, HIDDEN), lambda i: (0, 0)),
            pl.BlockSpec((_DENSE_BLK, HIDDEN), lambda i: (i, 0)),
        ],
        out_specs=[
            pl.BlockSpec((_DENSE_BLK, HIDDEN), lambda i: (i, 0)),
            pl.BlockSpec((_DENSE_BLK, HIDDEN), lambda i: (i, 0)),
            pl.BlockSpec((_DENSE_BLK, HIDDEN), lambda i: (i, 0)),
        ],
        out_shape=[
            jax.ShapeDtypeStruct((N_NODES, HIDDEN), jnp.float32),
            jax.ShapeDtypeStruct((N_NODES, HIDDEN), jnp.float32),
            jax.ShapeDtypeStruct((N_NODES, HIDDEN), jnp.float32),
        ],
    )(pa, dg, W_mu, b_mu, W_lv, b_lv, eps)


@functools.partial(
    pl.kernel,
    out_type=jax.ShapeDtypeStruct((N_EDGES,), jnp.float32),
    mesh=_mesh,
    scratch_types=(
        pltpu.VMEM((NCHUNK, K), jnp.int32),
        pltpu.VMEM((NCHUNK, K), jnp.int32),
        pltpu.VMEM((K, HIDDEN), jnp.float32),
        pltpu.VMEM((K, HIDDEN), jnp.float32),
        pltpu.VMEM((K, HIDDEN), jnp.float32),
        pltpu.VMEM((K, HIDDEN), jnp.float32),
        pltpu.VMEM((EPW,), jnp.float32),
        pltpu.SemaphoreType.DMA,
        pltpu.SemaphoreType.DMA,
    ),
    compiler_params=pltpu.CompilerParams(use_tc_tiling_on_sc=False, needs_layout_passes=False),
)
def _decoder(z_hbm, src_hbm, dst_hbm, recon_out,
             srcv, dstv, zs0, zs1, zd0, zd1, outv, sem0, sem1):
    c = lax.axis_index("c")
    s = lax.axis_index("s")
    w = s * NC + c
    base = pl.multiple_of(w * EPW, 8)
    pltpu.sync_copy(src_hbm.at[w], srcv)
    pltpu.sync_copy(dst_hbm.at[w], dstv)

    def _gather(ci, zsb, zdb, sem):
        pltpu.async_copy(z_hbm.at[srcv.at[ci]], zsb, sem)
        pltpu.async_copy(z_hbm.at[dstv.at[ci]], zdb, sem)

    def _wait(ci, zsb, zdb, sem):
        pltpu.make_async_copy(z_hbm.at[srcv.at[ci]], zsb, sem).wait()
        pltpu.make_async_copy(z_hbm.at[dstv.at[ci]], zdb, sem).wait()

    def _compute(ci, zsb, zdb):
        for g in range(K // 16):
            row = lax.iota(jnp.int32, 16) + (g * 16)
            acc = jnp.zeros((16,), jnp.float32)
            for dd in range(HIDDEN):
                col = jnp.full((16,), dd, jnp.int32)
                a = plsc.load_gather(zsb, [row, col])
                b = plsc.load_gather(zdb, [row, col])
                acc = acc + a * b
            outv[pl.ds(ci * K + g * 16, 16)] = 1.0 / (1.0 + jnp.exp(-acc))

    _gather(0, zs0, zd0, sem0)

    @pl.loop(0, (NCHUNK - 1) // 2)
    def _chunk(i):
        ci0 = i * 2
        ci1 = ci0 + 1
        _gather(ci1, zs1, zd1, sem1)
        _wait(ci0, zs0, zd0, sem0)
        _compute(ci0, zs0, zd0)
        _gather(ci0 + 2, zs0, zd0, sem0)
        _wait(ci1, zs1, zd1, sem1)
        _compute(ci1, zs1, zd1)

    _wait(NCHUNK - 1, zs0, zd0, sem0)
    _compute(NCHUNK - 1, zs0, zd0)
    pltpu.sync_copy(outv, recon_out.at[pl.ds(base, EPW)])


def kernel(x, edge_index, W_mu, b_mu, W_lv, b_lv):
    ei = edge_index.astype(jnp.int32)
    src_e = ei[0].reshape(NS, NCHUNK_E, K)
    dst_e = ei[1].reshape(NS, NCHUNK_E, K)
    src_d = ei[0].reshape(NW, NCHUNK, K)
    dst_d = ei[1].reshape(NW, NCHUNK, K)
    x2 = x.reshape(N_NODES, NC, HALF).transpose(1, 0, 2)
    zfeat = jnp.zeros((N_NODES, HALF), jnp.float32)
    zdeg = jnp.zeros((N_NODES, DEGW), jnp.float32)
    ones = jnp.ones((K, DEGW), jnp.float32)
    pa, dg = _encoder(x2, src_e, dst_e, zfeat, zdeg, ones)
    eps = jax.random.normal(jax.random.key(42), (N_NODES, HIDDEN), jnp.float32)
    mu, lv, z = _dense(pa, dg, W_mu, b_mu.reshape(1, HIDDEN),
                       W_lv, b_lv.reshape(1, HIDDEN), eps)
    recon = _decoder(z, src_d, dst_d)
    return (recon, mu, lv, z)
